# baseline (speedup 1.0000x reference)
import jax
import jax.numpy as jnp
from jax import lax
from jax.experimental import pallas as pl
from jax.experimental.pallas import tpu as pltpu
from jax.experimental.pallas import tpu_sc as plsc

N = 32768
D = 128
NC = 2
NS = 16
L = 16
CH = N // NS
GR = N // (NC * NS)
NB1 = 256
NB2 = 128
NBLK = CH // 128


def _body(values_hbm, keys_hbm, out_hbm,
          ck, dbuf, plb, posb, cnt, cq0, cq1, cq2, cq3, htot, gbuf, idxf,
          rb0, rb1, rb2, rb3, rb4, rb5,
          gs_s, a_s, ord_s,
          sem_sc, sg0, sg1, sg2, sg3, sg4, sg5,
          sw0, sw1, sw2, sw3, sw4, sw5):
  bufs = (rb0, rb1, rb2, rb3, rb4, rb5)
  gsems = (sg0, sg1, sg2, sg3, sg4, sg5)
  wsems = (sw0, sw1, sw2, sw3, sw4, sw5)
  s = lax.axis_index("s")
  c = lax.axis_index("c")

  probe, _ = plsc.scan_count(jnp.zeros((L,), jnp.int32))
  bias = jnp.min(probe)
  one_m_bias = 1 - bias

  wlt = [(jnp.int32(w) < s).astype(jnp.int32) for w in range(NS)]

  pltpu.sync_copy(keys_hbm.at[pl.ds(s * CH, CH)], ck)

  def counting_pass(pass1, dst_ref):
    nbins = NB1 if pass1 else NB2
    nvb = nbins // L
    tag = "p1" if pass1 else "p2"

    cqs = (cq0, cq1, cq2, cq3)
    for cq in cqs:
      for b in range(nvb):
        cq[pl.ds(b * L, L)] = jnp.zeros((L,), jnp.int32)
    scope_local = jax.named_scope(tag + "_local"); scope_local.__enter__()

    QV = CH // L // 4

    @pl.loop(jnp.int32(0), jnp.int32(QV))
    def _local(i):
      for q in range(4):
        sl = pl.ds(q * (CH // 4) + i * L, L)
        v = ck[sl]
        if pass1:
          idx = s * CH + (q * (CH // 4) + i * L) + lax.iota(jnp.int32, L)
          d = v & (NB1 - 1)
          ck[sl] = v * 32768 + idx
        else:
          d = lax.shift_right_logical(v, jnp.full((L,), 23, jnp.int32))
          ck[sl] = v & 32767
        run, last = plsc.scan_count(d)
        cur = plsc.load_gather(cqs[q], [d])
        dbuf[sl] = d
        plb[sl] = cur + run - bias
        plsc.addupdate_scatter(cqs[q], [d], run + one_m_bias, mask=last)

    scope_local.__exit__(None, None, None)
    with jax.named_scope(tag + "_exch"):
      for b in range(nvb):
        sl = pl.ds(b * L, L)
        htot[sl] = cq0[sl] + cq1[sl] + cq2[sl] + cq3[sl]
      pltpu.sync_copy(htot.at[pl.ds(0, nbins)], gs_s.at[s, pl.ds(0, nbins)])
      plsc.subcore_barrier()
      pltpu.sync_copy(gs_s, gbuf)
    scope_off = jax.named_scope(tag + "_off"); scope_off.__enter__()

    @pl.loop(jnp.int32(0), jnp.int32(nvb))
    def _sums(b):
      sl = pl.ds(b * L, L)
      tot = jnp.zeros((L,), jnp.int32)
      part = jnp.zeros((L,), jnp.int32)
      for w in range(NS):
        v = gbuf[w, sl]
        tot = tot + v
        part = part + v * wlt[w]
      dbuf[pl.ds(CH + b * L, L)] = tot
      cnt[sl] = part

    @pl.loop(jnp.int32(0), jnp.int32(nvb), init_carry=jnp.int32(0))
    def _scan(b, carry):
      sl = pl.ds(b * L, L)
      tot = dbuf[pl.ds(CH + b * L, L)]
      cnt[sl] = cnt[sl] + plsc.cumsum(tot) - tot + carry
      return carry + jnp.sum(tot, dtype=jnp.int32)

    for b in range(nvb):
      sl = pl.ds(b * L, L)
      s0 = cnt[sl]
      s1 = s0 + cq0[sl]
      s2 = s1 + cq1[sl]
      cq0[sl] = s1
      cq1[sl] = s2
      cq2[sl] = s2 + cq2[sl]
    starts = (cnt, cq0, cq1, cq2)

    scope_off.__exit__(None, None, None)
    scope_pos = jax.named_scope(tag + "_pos"); scope_pos.__enter__()
    descs = []
    for t in range(NBLK):
      for u in range(8):
        sl = pl.ds(t * 128 + u * L, L)
        d = dbuf[sl]
        posb[t, pl.ds(u * L, L)] = plsc.load_gather(starts[t // 4], [d]) + plb[sl]
      descs.append(pltpu.async_copy(
          ck.at[pl.ds(t * 128, 128)], dst_ref.at[posb.at[jnp.int32(t)]],
          sem_sc))
    for dsc in descs:
      dsc.wait()
    plsc.subcore_barrier()
    scope_pos.__exit__(None, None, None)

  counting_pass(True, a_s)
  pltpu.sync_copy(a_s.at[pl.ds(s * CH, CH)], ck)
  counting_pass(False, ord_s)

  scope_g = jax.named_scope("gath"); scope_g.__enter__()
  gbase = (s * NC + c) * GR
  pltpu.sync_copy(ord_s.at[pl.ds(gbase, GR)], idxf)

  nchunk = GR // 128
  NBUF = 6
  gdescs = [None] * NBUF
  wdescs = [None] * NBUF
  for r in range(NBUF - 1):
    gdescs[r] = pltpu.async_copy(
        values_hbm.at[idxf.at[pl.ds(r * 128, 128)]], bufs[r], gsems[r])
  for r in range(nchunk):
    b = r % NBUF
    gdescs[b].wait()
    wdescs[b] = pltpu.async_copy(
        bufs[b], out_hbm.at[pl.ds(gbase + r * 128, 128)], wsems[b])
    nr = r + NBUF - 1
    if nr < nchunk:
      bb = nr % NBUF
      if wdescs[bb] is not None:
        wdescs[bb].wait()
      gdescs[bb] = pltpu.async_copy(
          values_hbm.at[idxf.at[pl.ds(nr * 128, 128)]], bufs[bb], gsems[bb])
  for b in range(NBUF):
    wdescs[b].wait()
  scope_g.__exit__(None, None, None)


@jax.jit
def kernel(values, sample_ids):
  keys32 = sample_ids.astype(jnp.int32)
  mesh = plsc.VectorSubcoreMesh(
      core_axis_name="c", subcore_axis_name="s",
      num_cores=NC, num_subcores=NS)
  fn = pl.kernel(
      _body,
      out_type=jax.ShapeDtypeStruct((N, D), jnp.float32),
      mesh=mesh,
      scratch_types=[
          pltpu.VMEM((CH,), jnp.int32),
          pltpu.VMEM((CH + NB1,), jnp.int32),
          pltpu.VMEM((CH,), jnp.int32),
          pltpu.VMEM((NBLK, 128), jnp.int32),
          pltpu.VMEM((NB1,), jnp.int32),
          pltpu.VMEM((NB1,), jnp.int32),
          pltpu.VMEM((NB1,), jnp.int32),
          pltpu.VMEM((NB1,), jnp.int32),
          pltpu.VMEM((NB1,), jnp.int32),
          pltpu.VMEM((NB1,), jnp.int32),
          pltpu.VMEM((NS, NB1), jnp.int32),
          pltpu.VMEM((GR,), jnp.int32),
          pltpu.VMEM((128, D), jnp.float32),
          pltpu.VMEM((128, D), jnp.float32),
          pltpu.VMEM((128, D), jnp.float32),
          pltpu.VMEM((128, D), jnp.float32),
          pltpu.VMEM((128, D), jnp.float32),
          pltpu.VMEM((128, D), jnp.float32),
          pltpu.VMEM_SHARED((NS, NB1), jnp.int32),
          pltpu.VMEM_SHARED((N,), jnp.int32),
          pltpu.VMEM_SHARED((N,), jnp.int32),
          pltpu.SemaphoreType.DMA,
          pltpu.SemaphoreType.DMA,
          pltpu.SemaphoreType.DMA,
          pltpu.SemaphoreType.DMA,
          pltpu.SemaphoreType.DMA,
          pltpu.SemaphoreType.DMA,
          pltpu.SemaphoreType.DMA,
          pltpu.SemaphoreType.DMA,
          pltpu.SemaphoreType.DMA,
          pltpu.SemaphoreType.DMA,
          pltpu.SemaphoreType.DMA,
          pltpu.SemaphoreType.DMA,
          pltpu.SemaphoreType.DMA,
      ],
      compiler_params=pltpu.CompilerParams(needs_layout_passes=False),
      name="densify_sc",
  )
  return fn(values, keys32)


# speedup vs baseline: 1.0192x; 1.0192x over previous
import jax
import jax.numpy as jnp
from jax import lax
from jax.experimental import pallas as pl
from jax.experimental.pallas import tpu as pltpu
from jax.experimental.pallas import tpu_sc as plsc

N = 32768
D = 128
NC = 2
NS = 16
L = 16
CH = N // NS
GR = N // (NC * NS)
NB1 = 256
NB2 = 128
NBLK = CH // 128
NBUF = 6


def _body(values_hbm, keys_hbm, out_hbm,
          ck, dbuf, plb, posb, cnt, gbuf, idxf,
          rb0, rb1, rb2, rb3, rb4, rb5,
          gs_s, a_s, ord_s,
          sem_sc, sg0, sg1, sg2, sg3, sg4, sg5,
          sw0, sw1, sw2, sw3, sw4, sw5):
  bufs = (rb0, rb1, rb2, rb3, rb4, rb5)
  gsems = (sg0, sg1, sg2, sg3, sg4, sg5)
  wsems = (sw0, sw1, sw2, sw3, sw4, sw5)
  s = lax.axis_index("s")
  c = lax.axis_index("c")

  probe, _ = plsc.scan_count(jnp.zeros((L,), jnp.int32))
  bias = jnp.min(probe)
  one_m_bias = 1 - bias

  wlt = [(jnp.int32(w) < s).astype(jnp.int32) for w in range(NS)]

  pltpu.sync_copy(keys_hbm.at[pl.ds(s * CH, CH)], ck)

  def counting_pass(pass1, dst_ref):
    nbins = NB1 if pass1 else NB2
    nvb = nbins // L

    for b in range(nvb):
      cnt[pl.ds(b * L, L)] = jnp.zeros((L,), jnp.int32)

    @pl.loop(jnp.int32(0), jnp.int32(CH // L))
    def _local(i):
      sl = pl.ds(i * L, L)
      v = ck[sl]
      if pass1:
        idx = s * CH + i * L + lax.iota(jnp.int32, L)
        d = v & (NB1 - 1)
        ck[sl] = v * 32768 + idx
      else:
        d = lax.shift_right_logical(v, jnp.full((L,), 23, jnp.int32))
        ck[sl] = v & 32767
      run, last = plsc.scan_count(d)
      cur = plsc.load_gather(cnt, [d])
      dbuf[sl] = d
      plb[sl] = cur + run - bias
      plsc.addupdate_scatter(cnt, [d], run + one_m_bias, mask=last)

    pltpu.sync_copy(cnt.at[pl.ds(0, nbins)], gs_s.at[s, pl.ds(0, nbins)])
    plsc.subcore_barrier()
    pltpu.sync_copy(gs_s, gbuf)

    @pl.loop(jnp.int32(0), jnp.int32(nvb))
    def _sums(b):
      sl = pl.ds(b * L, L)
      tot = jnp.zeros((L,), jnp.int32)
      part = jnp.zeros((L,), jnp.int32)
      for w in range(NS):
        v = gbuf[w, sl]
        tot = tot + v
        part = part + v * wlt[w]
      dbuf[pl.ds(CH + b * L, L)] = tot
      cnt[sl] = part

    @pl.loop(jnp.int32(0), jnp.int32(nvb), init_carry=jnp.int32(0))
    def _scan(b, carry):
      sl = pl.ds(b * L, L)
      tot = dbuf[pl.ds(CH + b * L, L)]
      cnt[sl] = cnt[sl] + plsc.cumsum(tot) - tot + carry
      return carry + jnp.sum(tot, dtype=jnp.int32)

    descs = []
    for t in range(NBLK):
      for u in range(8):
        sl = pl.ds(t * 128 + u * L, L)
        d = dbuf[sl]
        posb[t, pl.ds(u * L, L)] = plsc.load_gather(cnt, [d]) + plb[sl]
      descs.append(pltpu.async_copy(
          ck.at[pl.ds(t * 128, 128)], dst_ref.at[posb.at[jnp.int32(t)]],
          sem_sc))
    for dsc in descs:
      dsc.wait()
    plsc.subcore_barrier()

  counting_pass(True, a_s)
  pltpu.sync_copy(a_s.at[pl.ds(s * CH, CH)], ck)
  counting_pass(False, ord_s)

  gbase = (s * NC + c) * GR
  pltpu.sync_copy(ord_s.at[pl.ds(gbase, GR)], idxf)

  nchunk = GR // 128
  gdescs = [None] * NBUF
  wdescs = [None] * NBUF
  for r in range(NBUF - 1):
    gdescs[r] = pltpu.async_copy(
        values_hbm.at[idxf.at[pl.ds(r * 128, 128)]], bufs[r], gsems[r])
  for r in range(nchunk):
    b = r % NBUF
    gdescs[b].wait()
    wdescs[b] = pltpu.async_copy(
        bufs[b], out_hbm.at[pl.ds(gbase + r * 128, 128)], wsems[b])
    nr = r + NBUF - 1
    if nr < nchunk:
      bb = nr % NBUF
      if wdescs[bb] is not None:
        wdescs[bb].wait()
      gdescs[bb] = pltpu.async_copy(
          values_hbm.at[idxf.at[pl.ds(nr * 128, 128)]], bufs[bb], gsems[bb])
  for b in range(NBUF):
    if wdescs[b] is not None:
      wdescs[b].wait()


@jax.jit
def kernel(values, sample_ids):
  keys32 = sample_ids.astype(jnp.int32)
  mesh = plsc.VectorSubcoreMesh(
      core_axis_name="c", subcore_axis_name="s",
      num_cores=NC, num_subcores=NS)
  fn = pl.kernel(
      _body,
      out_type=jax.ShapeDtypeStruct((N, D), jnp.float32),
      mesh=mesh,
      scratch_types=(
          [
              pltpu.VMEM((CH,), jnp.int32),
              pltpu.VMEM((CH + NB1,), jnp.int32),
              pltpu.VMEM((CH,), jnp.int32),
              pltpu.VMEM((NBLK, 128), jnp.int32),
              pltpu.VMEM((NB1,), jnp.int32),
              pltpu.VMEM((NS, NB1), jnp.int32),
              pltpu.VMEM((GR,), jnp.int32),
          ]
          + [pltpu.VMEM((128, D), jnp.float32)] * NBUF
          + [
              pltpu.VMEM_SHARED((NS, NB1), jnp.int32),
              pltpu.VMEM_SHARED((N,), jnp.int32),
              pltpu.VMEM_SHARED((N,), jnp.int32),
          ]
          + [pltpu.SemaphoreType.DMA] * (1 + 2 * NBUF)
      ),
      compiler_params=pltpu.CompilerParams(needs_layout_passes=False),
      name="densify_sc",
  )
  return fn(values, keys32)
